# Initial kernel scaffold; baseline (speedup 1.0000x reference)
import jax
import jax.numpy as jnp
from jax.experimental import pallas as pl


def kernel(x, W1, W2):
    raise NotImplementedError("write your pallas kernel here")


# speedup vs baseline: 1.9259x; 1.9259x over previous
import jax
import jax.numpy as jnp
from jax.experimental import pallas as pl

B, S, D = 4, 2048, 2048
H, E = 8, 512
TOPK = 8
EPS = 1e-5

M = B * S
BM = 1024
NM = M // BM


def _mm_stats_kernel(x_ref, w_ref, z_ref, st_ref):
    z = jax.lax.dot_general(
        x_ref[...], w_ref[...], (((1,), (1,)), ((), ())),
        preferred_element_type=jnp.float32)
    z_ref[...] = z
    s = jnp.sum(z)
    q = jnp.sum(z * z)
    rows = jax.lax.broadcasted_iota(jnp.int32, (8, 128), 0)
    vals = jnp.where(rows == 0, s, jnp.where(rows == 1, q, 0.0))
    st_ref[...] = vals.reshape(1, 1, 8, 128)


def _mm_stats(x2, w):
    return pl.pallas_call(
        _mm_stats_kernel,
        grid=(NM, H),
        in_specs=[
            pl.BlockSpec((BM, D), lambda m, h: (m, 0)),
            pl.BlockSpec((E, D), lambda m, h: (h, 0)),
        ],
        out_specs=[
            pl.BlockSpec((BM, E), lambda m, h: (m, h)),
            pl.BlockSpec((1, 1, 8, 128), lambda m, h: (m, h, 0, 0)),
        ],
        out_shape=[
            jax.ShapeDtypeStruct((M, H * E), jnp.float32),
            jax.ShapeDtypeStruct((NM, H, 8, 128), jnp.float32),
        ],
    )(x2, w)


def _thresholds(st):
    s = st[:, :, 0, 0].sum(axis=0)
    q = st[:, :, 1, 0].sum(axis=0)
    n = float(M * E)
    mean = s / n
    var = q / n - mean * mean
    p = 1.0 - float(TOPK) / float(E)
    sigma = jnp.sqrt(2.0) * jax.scipy.special.erfinv(2.0 * p - 1.0)
    c = sigma * jnp.sqrt(var + EPS) + mean
    return jnp.broadcast_to(c[:, None, None], (H, 8, 128))


def _softmax_kernel(z1_ref, z2_ref, c1_ref, c2_ref, o1_ref, o2_ref):
    for z_ref, c_ref, o_ref in ((z1_ref, c1_ref, o1_ref),
                                (z2_ref, c2_ref, o2_ref)):
        z = z_ref[...]
        c = c_ref[0, 0, 0]
        rowmax = jnp.max(z, axis=1, keepdims=True)
        t = jnp.minimum(rowmax, c)
        e = jnp.where(z >= t, jnp.exp(z - rowmax), 0.0)
        den = jnp.sum(e, axis=1, keepdims=True)
        o_ref[...] = e / den


def _masked_softmax(z1, z2, c1, c2):
    zspec = pl.BlockSpec((BM, E), lambda m, h: (m, h))
    cspec = pl.BlockSpec((1, 8, 128), lambda m, h: (h, 0, 0))
    oshape = jax.ShapeDtypeStruct((M, H * E), jnp.float32)
    return pl.pallas_call(
        _softmax_kernel,
        grid=(NM, H),
        in_specs=[zspec, zspec, cspec, cspec],
        out_specs=[zspec, zspec],
        out_shape=[oshape, oshape],
    )(z1, z2, c1, c2)


def kernel(x, W1, W2):
    x2 = x.reshape(M, D)
    z1, st1 = _mm_stats(x2, W1)
    z2, st2 = _mm_stats(x2, W2)
    c1 = _thresholds(st1)
    c2 = _thresholds(st2)
    g1, g2 = _masked_softmax(z1, z2, c1, c2)
    return g1.reshape(B, S, H, E), g2.reshape(B, S, H, E)


# speedup vs baseline: 2.9414x; 1.5273x over previous
import jax
import jax.numpy as jnp
from jax.experimental import pallas as pl

B, S, D = 4, 2048, 2048
H, E = 8, 512
TOPK = 8
EPS = 1e-5

M = B * S
HE = H * E
BM1 = 256
NM1 = M // BM1
BM2 = 512
NM2 = M // BM2
SB2 = S // BM2


def _mm_stats_kernel(x_ref, w_ref, z_ref, st_ref):
    z = jax.lax.dot_general(
        x_ref[...], w_ref[...], (((1,), (1,)), ((), ())),
        preferred_element_type=jnp.float32)
    z_ref[...] = z
    rows = jax.lax.broadcasted_iota(jnp.int32, (8, 128), 0)
    cols = jax.lax.broadcasted_iota(jnp.int32, (8, 128), 1)
    acc = jnp.zeros((8, 128), jnp.float32)
    for h in range(H):
        zh = z[:, h * E:(h + 1) * E]
        s = jnp.sum(zh)
        q = jnp.sum(zh * zh)
        acc = acc + jnp.where((rows == h) & (cols == 0), s, 0.0)
        acc = acc + jnp.where((rows == h) & (cols == 1), q, 0.0)
    st_ref[...] = acc.reshape(1, 8, 128)


def _mm_stats(x2, w):
    return pl.pallas_call(
        _mm_stats_kernel,
        grid=(NM1,),
        in_specs=[
            pl.BlockSpec((BM1, D), lambda m: (m, 0)),
            pl.BlockSpec((HE, D), lambda m: (0, 0)),
        ],
        out_specs=[
            pl.BlockSpec((BM1, HE), lambda m: (m, 0)),
            pl.BlockSpec((1, 8, 128), lambda m: (m, 0, 0)),
        ],
        out_shape=[
            jax.ShapeDtypeStruct((M, HE), jnp.float32),
            jax.ShapeDtypeStruct((NM1, 8, 128), jnp.float32),
        ],
    )(x2, w)


def _thresholds(st):
    s = st[:, :, 0].sum(axis=0)
    q = st[:, :, 1].sum(axis=0)
    n = float(M * E)
    mean = s / n
    var = q / n - mean * mean
    p = 1.0 - float(TOPK) / float(E)
    sigma = jnp.sqrt(2.0) * jax.scipy.special.erfinv(2.0 * p - 1.0)
    c = sigma * jnp.sqrt(var + EPS) + mean
    return jnp.broadcast_to(c[:, None], (H, 128))


def _softmax_kernel(z_ref, c_ref, o_ref):
    z4 = z_ref[...].reshape(BM2, H, E)
    c = c_ref[...][:, :1]
    rowmax = jnp.max(z4, axis=2, keepdims=True)
    t = jnp.minimum(rowmax, c)
    e = jnp.where(z4 >= t, jnp.exp(z4 - rowmax), 0.0)
    den = jnp.sum(e, axis=2, keepdims=True)
    o_ref[...] = (e / den).reshape(1, BM2, H, E)


def _masked_softmax(z, c):
    return pl.pallas_call(
        _softmax_kernel,
        grid=(NM2,),
        in_specs=[
            pl.BlockSpec((BM2, HE), lambda m: (m, 0)),
            pl.BlockSpec((H, 128), lambda m: (0, 0)),
        ],
        out_specs=pl.BlockSpec(
            (1, BM2, H, E), lambda m: (m // SB2, m % SB2, 0, 0)),
        out_shape=jax.ShapeDtypeStruct((B, S, H, E), jnp.float32),
    )(z, c)


def kernel(x, W1, W2):
    x2 = x.reshape(M, D)
    z1, st1 = _mm_stats(x2, W1)
    z2, st2 = _mm_stats(x2, W2)
    c1 = _thresholds(st1)
    c2 = _thresholds(st2)
    g1 = _masked_softmax(z1, c1)
    g2 = _masked_softmax(z2, c2)
    return g1, g2
